# Initial kernel scaffold; baseline (speedup 1.0000x reference)
import functools

import jax
import jax.numpy as jnp
from jax import lax
from jax.experimental import pallas as pl
from jax.experimental.pallas import tpu as pltpu
from jax.experimental.pallas import tpu_sc as plsc

N = 10000
E = 320000
D = 128
NC = 2
NS = 16
K = 200
EPT = E // (NC * NS)
NB = EPT // K
RPT = N // NS


def _sc_body(x_hbm, src_hbm, dst_hbm, w_hbm, zeros_hbm, out_hbm,
             src_v, dst_v, w_v, rows_v, acc, sem):
    cid = lax.axis_index("c")
    sid = lax.axis_index("s")

    pltpu.sync_copy(zeros_hbm.at[pl.ds(sid * RPT, RPT)],
                    acc.at[pl.ds(sid * RPT, RPT)])
    plsc.subcore_barrier()

    def batch_body(b, _):
        ebase = cid * (E // NC) + sid * EPT + b * K
        pltpu.sync_copy(src_hbm.at[pl.ds(ebase, K)], src_v)
        pltpu.sync_copy(dst_hbm.at[pl.ds(ebase, K)], dst_v)
        pltpu.sync_copy(w_hbm.at[pl.ds(ebase, K)], w_v)
        pltpu.async_copy(x_hbm.at[src_v], rows_v, sem).wait()

        def edge_body(i, _):
            wb = plsc.load_gather(w_v, (jnp.full((16,), 0, jnp.int32) + i,))
            for j in range(D // 16):
                sl = pl.ds(j * 16, 16)
                rows_v[i, sl] = rows_v[i, sl] * wb
            return 0

        lax.fori_loop(0, K, edge_body, 0, unroll=2)
        pltpu.sync_copy(rows_v, acc.at[dst_v], add=True)
        return 0

    lax.fori_loop(0, NB, batch_body, 0)
    plsc.subcore_barrier()
    pltpu.sync_copy(acc.at[pl.ds(sid * RPT, RPT)],
                    out_hbm.at[cid, pl.ds(sid * RPT, RPT)])


_sc_aggregate = pl.kernel(
    _sc_body,
    out_type=jax.ShapeDtypeStruct((NC, N, D), jnp.float32),
    mesh=plsc.VectorSubcoreMesh(core_axis_name="c", subcore_axis_name="s"),
    scratch_types=[
        pltpu.VMEM((K,), jnp.int32),
        pltpu.VMEM((K,), jnp.int32),
        pltpu.VMEM((K,), jnp.float32),
        pltpu.VMEM((K, D), jnp.float32),
        pltpu.VMEM_SHARED((N, D), jnp.float32),
        pltpu.SemaphoreType.DMA,
    ],
)


def _mm_body(pa_ref, pb_ref, w_ref, b_ref, o_ref):
    acc = pa_ref[...] + pb_ref[...]
    o_ref[...] = (
        jnp.dot(acc, w_ref[...], preferred_element_type=jnp.float32)
        + b_ref[...]
    )


_BM = 1250


def _tc_matmul(parts, weights, bias2d):
    return pl.pallas_call(
        _mm_body,
        out_shape=jax.ShapeDtypeStruct((N, D), jnp.float32),
        grid=(N // _BM,),
        in_specs=[
            pl.BlockSpec((_BM, D), lambda i: (i, 0)),
            pl.BlockSpec((_BM, D), lambda i: (i, 0)),
            pl.BlockSpec((D, D), lambda i: (0, 0)),
            pl.BlockSpec((1, D), lambda i: (0, 0)),
        ],
        out_specs=pl.BlockSpec((_BM, D), lambda i: (i, 0)),
    )(parts[0], parts[1], weights, bias2d)


def kernel(feature_map, edge_index, edge_weight, weights, bias):
    src = edge_index[0].astype(jnp.int32)
    dst = edge_index[1].astype(jnp.int32)
    zeros = jnp.zeros((N, D), jnp.float32)
    parts = _sc_aggregate(feature_map, src, dst, edge_weight, zeros)
    return _tc_matmul(parts, weights, bias.reshape(1, D))


# speedup vs baseline: 4.4292x; 4.4292x over previous
import functools

import jax
import jax.numpy as jnp
from jax import lax
from jax.experimental import pallas as pl
from jax.experimental.pallas import tpu as pltpu
from jax.experimental.pallas import tpu_sc as plsc

N = 10000
E = 320000
D = 128
NC = 2
NS = 16
K = 80
EPT = E // (NC * NS)
NB = EPT // K
ZR = 624


def _sc_body(x_hbm, src_hbm, dst_hbm, w_hbm, zeros_hbm, out_hbm,
             src_v, dst_v, w_v, rows_v, acc, sem):
    cid = lax.axis_index("c")
    sid = lax.axis_index("s")

    @pl.when(sid < NS - 1)
    def _():
        pltpu.sync_copy(zeros_hbm.at[pl.ds(sid * ZR, ZR)],
                        acc.at[pl.ds(sid * ZR, ZR)])

    @pl.when(sid == NS - 1)
    def _():
        pltpu.sync_copy(zeros_hbm.at[pl.ds((NS - 1) * ZR, N - (NS - 1) * ZR)],
                        acc.at[pl.ds((NS - 1) * ZR, N - (NS - 1) * ZR)])

    plsc.subcore_barrier()

    def batch_body(b, _):
        ebase = cid * (E // NC) + sid * EPT + b * K
        pltpu.sync_copy(src_hbm.at[pl.ds(ebase, K)], src_v)
        pltpu.sync_copy(dst_hbm.at[pl.ds(ebase, K)], dst_v)
        pltpu.sync_copy(w_hbm.at[pl.ds(ebase, K)], w_v)
        pltpu.async_copy(x_hbm.at[src_v], rows_v, sem).wait()

        def group_body(g, _):
            w16 = w_v[pl.ds(g * 16, 16)]
            for e in range(16):
                wb = lax.gather(
                    w16,
                    jnp.full((16, 1), e, jnp.int32),
                    lax.GatherDimensionNumbers(
                        offset_dims=(), collapsed_slice_dims=(0,),
                        start_index_map=(0,)),
                    slice_sizes=(1,),
                    mode=lax.GatherScatterMode.PROMISE_IN_BOUNDS,
                )
                row = g * 16 + e
                for j in range(D // 16):
                    sl = pl.ds(j * 16, 16)
                    rows_v[row, sl] = rows_v[row, sl] * wb
            return 0

        lax.fori_loop(0, K // 16, group_body, 0)
        pltpu.sync_copy(rows_v, acc.at[dst_v], add=True)
        return 0

    lax.fori_loop(0, NB, batch_body, 0)
    plsc.subcore_barrier()

    @pl.when(sid < NS - 1)
    def _():
        pltpu.sync_copy(acc.at[pl.ds(sid * ZR, ZR)],
                        out_hbm.at[cid, pl.ds(sid * ZR, ZR)])

    @pl.when(sid == NS - 1)
    def _():
        pltpu.sync_copy(acc.at[pl.ds((NS - 1) * ZR, N - (NS - 1) * ZR)],
                        out_hbm.at[cid, pl.ds((NS - 1) * ZR, N - (NS - 1) * ZR)])


_sc_aggregate = pl.kernel(
    _sc_body,
    out_type=jax.ShapeDtypeStruct((NC, N, D), jnp.float32),
    mesh=plsc.VectorSubcoreMesh(core_axis_name="c", subcore_axis_name="s"),
    scratch_types=[
        pltpu.VMEM((K,), jnp.int32),
        pltpu.VMEM((K,), jnp.int32),
        pltpu.VMEM((K,), jnp.float32),
        pltpu.VMEM((K, D), jnp.float32),
        pltpu.VMEM_SHARED((N, D), jnp.float32),
        pltpu.SemaphoreType.DMA,
    ],
)


def _mm_body(pa_ref, pb_ref, w_ref, b_ref, o_ref):
    acc = pa_ref[...] + pb_ref[...]
    o_ref[...] = (
        jnp.dot(acc, w_ref[...], preferred_element_type=jnp.float32)
        + b_ref[...]
    )


_BM = 1000


def _tc_matmul(parts, weights, bias2d):
    return pl.pallas_call(
        _mm_body,
        out_shape=jax.ShapeDtypeStruct((N, D), jnp.float32),
        grid=(N // _BM,),
        in_specs=[
            pl.BlockSpec((_BM, D), lambda i: (i, 0)),
            pl.BlockSpec((_BM, D), lambda i: (i, 0)),
            pl.BlockSpec((D, D), lambda i: (0, 0)),
            pl.BlockSpec((1, D), lambda i: (0, 0)),
        ],
        out_specs=pl.BlockSpec((_BM, D), lambda i: (i, 0)),
    )(parts[0], parts[1], weights, bias2d)


def kernel(feature_map, edge_index, edge_weight, weights, bias):
    src = edge_index[0].astype(jnp.int32)
    dst = edge_index[1].astype(jnp.int32)
    zeros = jnp.zeros((N, D), jnp.float32)
    parts = _sc_aggregate(feature_map, src, dst, edge_weight, zeros)
    return _tc_matmul(parts, weights, bias.reshape(1, D))
